# baseline (speedup 1.0000x reference)
import functools

import jax
import jax.numpy as jnp
from jax.experimental import pallas as pl
from jax.experimental.pallas import tpu as pltpu

_K = 20
_EPS = 1e-5
_NEG = float("-inf")


def _knn_edge_kernel(xfull_ref, xtile_ref, ed_ref, *, T, N, K):
    xf = xfull_ref[0]
    xi = xtile_ref[0]
    xx = jnp.sum(xf * xf, axis=0)
    xxi = jnp.sum(xi * xi, axis=0)
    ip = jax.lax.dot_general(xi, xf, (((0,), (0,)), ((), ())),
                             preferred_element_type=jnp.float32)
    dist = 2.0 * ip - xxi[:, None] - xx[None, :]
    iota = jax.lax.broadcasted_iota(jnp.int32, (T, N), 1)
    parts = []
    for _ in range(K):
        a = jnp.argmax(dist, axis=1)
        ohm = iota == a[:, None]
        ohf = ohm.astype(jnp.float32)
        xj = jax.lax.dot_general(xf, ohf, (((1,), (1,)), ((), ())),
                                 preferred_element_type=jnp.float32)
        parts.append(xj - xi)
        dist = jnp.where(ohm, _NEG, dist)
    ed_ref[0] = jnp.concatenate(parts, axis=0)


def _stats1_kernel(ed_ref, xtile_ref, w1_ref, s_ref, q_ref, *, K):
    @pl.when((pl.program_id(0) == 0) & (pl.program_id(1) == 0))
    def _init():
        s_ref[...] = jnp.zeros_like(s_ref)
        q_ref[...] = jnp.zeros_like(q_ref)

    ed = ed_ref[0]
    xi = xtile_ref[0]
    c = xi.shape[0]
    w1a = w1_ref[:, :c]
    w1b = w1_ref[:, c:]
    u = jnp.dot(w1b, xi, preferred_element_type=jnp.float32)
    s = jnp.zeros((s_ref.shape[0],), jnp.float32)
    q = jnp.zeros((s_ref.shape[0],), jnp.float32)
    for k in range(K):
        h = jnp.dot(w1a, ed[k * c:(k + 1) * c, :],
                    preferred_element_type=jnp.float32) + u
        s = s + jnp.sum(h, axis=1)
        q = q + jnp.sum(h * h, axis=1)
    s_ref[...] += s[:, None]
    q_ref[...] += q[:, None]


def _affine(s, q, g, b, count):
    mu = s / count
    var = q / count - mu * mu
    sc = g * jax.lax.rsqrt(var + _EPS)
    sh = b - mu * sc
    return sc, sh


def _stats2_kernel(ed_ref, xtile_ref, w1_ref, w2_ref, s1_ref, q1_ref,
                   g1_ref, b1_ref, s_ref, q_ref, *, K, count):
    @pl.when((pl.program_id(0) == 0) & (pl.program_id(1) == 0))
    def _init():
        s_ref[...] = jnp.zeros_like(s_ref)
        q_ref[...] = jnp.zeros_like(q_ref)

    ed = ed_ref[0]
    xi = xtile_ref[0]
    c = xi.shape[0]
    w1a = w1_ref[:, :c]
    w1b = w1_ref[:, c:]
    sc1, sh1 = _affine(s1_ref[...], q1_ref[...], g1_ref[...], b1_ref[...],
                       count)
    u = jnp.dot(w1b, xi, preferred_element_type=jnp.float32)
    s = jnp.zeros((s_ref.shape[0],), jnp.float32)
    q = jnp.zeros((s_ref.shape[0],), jnp.float32)
    for k in range(K):
        h = jnp.dot(w1a, ed[k * c:(k + 1) * c, :],
                    preferred_element_type=jnp.float32) + u
        r = jnp.maximum(h * sc1 + sh1, 0.0)
        h2 = jnp.dot(w2_ref[...], r, preferred_element_type=jnp.float32)
        s = s + jnp.sum(h2, axis=1)
        q = q + jnp.sum(h2 * h2, axis=1)
    s_ref[...] += s[:, None]
    q_ref[...] += q[:, None]


def _final_kernel(ed_ref, xtile_ref, w1_ref, w2_ref, s1_ref, q1_ref,
                  g1_ref, b1_ref, s2_ref, q2_ref, g2_ref, b2_ref,
                  out_ref, *, K, count):
    ed = ed_ref[0]
    xi = xtile_ref[0]
    c = xi.shape[0]
    w1a = w1_ref[:, :c]
    w1b = w1_ref[:, c:]
    sc1, sh1 = _affine(s1_ref[...], q1_ref[...], g1_ref[...], b1_ref[...],
                       count)
    sc2, sh2 = _affine(s2_ref[...], q2_ref[...], g2_ref[...], b2_ref[...],
                       count)
    u = jnp.dot(w1b, xi, preferred_element_type=jnp.float32)
    acc = jnp.full((out_ref.shape[1], xi.shape[1]), _NEG, jnp.float32)
    for k in range(K):
        h = jnp.dot(w1a, ed[k * c:(k + 1) * c, :],
                    preferred_element_type=jnp.float32) + u
        r = jnp.maximum(h * sc1 + sh1, 0.0)
        h2 = jnp.dot(w2_ref[...], r, preferred_element_type=jnp.float32)
        acc = jnp.maximum(acc, h2 * sc2 + sh2)
    out_ref[0] = jnp.maximum(acc, 0.0)


def kernel(x, W1, g1, b1, W2, g2, b2):
    B, C, N = x.shape
    O = W1.shape[0]
    K = _K
    T1 = 256
    T2 = 512
    par = pltpu.CompilerParams(dimension_semantics=("parallel", "parallel"))
    f32 = jnp.float32
    g1c = g1.reshape(O, 1)
    b1c = b1.reshape(O, 1)
    g2c = g2.reshape(O, 1)
    b2c = b2.reshape(O, 1)
    count = float(B * N * K)

    ed = pl.pallas_call(
        functools.partial(_knn_edge_kernel, T=T1, N=N, K=K),
        grid=(B, N // T1),
        in_specs=[
            pl.BlockSpec((1, C, N), lambda b, t: (b, 0, 0)),
            pl.BlockSpec((1, C, T1), lambda b, t: (b, 0, t)),
        ],
        out_specs=pl.BlockSpec((1, C * K, T1), lambda b, t: (b, 0, t)),
        out_shape=jax.ShapeDtypeStruct((B, C * K, N), f32),
        compiler_params=par,
    )(x, x)

    vec_spec = pl.BlockSpec((O, 1), lambda b, t: (0, 0))
    w1_spec = pl.BlockSpec((O, 2 * C), lambda b, t: (0, 0))
    w2_spec = pl.BlockSpec((O, O), lambda b, t: (0, 0))
    ed_spec = pl.BlockSpec((1, C * K, T2), lambda b, t: (b, 0, t))
    xt_spec = pl.BlockSpec((1, C, T2), lambda b, t: (b, 0, t))

    s1, q1 = pl.pallas_call(
        functools.partial(_stats1_kernel, K=K),
        grid=(B, N // T2),
        in_specs=[ed_spec, xt_spec, w1_spec],
        out_specs=[vec_spec, vec_spec],
        out_shape=[jax.ShapeDtypeStruct((O, 1), f32)] * 2,
    )(ed, x, W1)

    s2, q2 = pl.pallas_call(
        functools.partial(_stats2_kernel, K=K, count=count),
        grid=(B, N // T2),
        in_specs=[ed_spec, xt_spec, w1_spec, w2_spec,
                  vec_spec, vec_spec, vec_spec, vec_spec],
        out_specs=[vec_spec, vec_spec],
        out_shape=[jax.ShapeDtypeStruct((O, 1), f32)] * 2,
    )(ed, x, W1, W2, s1, q1, g1c, b1c)

    out = pl.pallas_call(
        functools.partial(_final_kernel, K=K, count=count),
        grid=(B, N // T2),
        in_specs=[ed_spec, xt_spec, w1_spec, w2_spec,
                  vec_spec, vec_spec, vec_spec, vec_spec,
                  vec_spec, vec_spec, vec_spec, vec_spec],
        out_specs=pl.BlockSpec((1, O, T2), lambda b, t: (b, 0, t)),
        out_shape=jax.ShapeDtypeStruct((B, O, N), f32),
        compiler_params=par,
    )(ed, x, W1, W2, s1, q1, g1c, b1c, s2, q2, g2c, b2c)

    return out


# speedup vs baseline: 1.0561x; 1.0561x over previous
import functools

import jax
import jax.numpy as jnp
from jax.experimental import pallas as pl
from jax.experimental.pallas import tpu as pltpu

_K = 20
_EPS = 1e-5
_NEG = float("-inf")


def _knn_edge_kernel(xfull_ref, xtile_ref, ed_ref, *, T, N, K):
    xf = xfull_ref[0]
    xi = xtile_ref[0]
    xx = jnp.sum(xf * xf, axis=0)
    xia = jnp.concatenate([2.0 * xi, jnp.full((1, xi.shape[1]), -1.0,
                                              jnp.float32)], axis=0)
    xfa = jnp.concatenate([xf, xx[None, :]], axis=0)
    dist = jax.lax.dot_general(xia, xfa, (((0,), (0,)), ((), ())),
                               preferred_element_type=jnp.float32)
    iota = jax.lax.broadcasted_iota(jnp.int32, (T, N), 1)
    parts = []
    for _ in range(K):
        a = jnp.argmax(dist, axis=1)
        ohm = iota == a[:, None]
        ohf = ohm.astype(jnp.float32)
        xj = jax.lax.dot_general(xf, ohf, (((1,), (1,)), ((), ())),
                                 preferred_element_type=jnp.float32)
        parts.append(xj - xi)
        dist = jnp.where(ohm, _NEG, dist)
    ed_ref[0] = jnp.concatenate(parts, axis=0)


def _stats1_kernel(ed_ref, xtile_ref, w1_ref, s_ref, q_ref, *, K):
    @pl.when((pl.program_id(0) == 0) & (pl.program_id(1) == 0))
    def _init():
        s_ref[...] = jnp.zeros_like(s_ref)
        q_ref[...] = jnp.zeros_like(q_ref)

    ed = ed_ref[0]
    xi = xtile_ref[0]
    c = xi.shape[0]
    w1a = w1_ref[:, :c]
    w1b = w1_ref[:, c:]
    u = jnp.dot(w1b, xi, preferred_element_type=jnp.float32)
    s = jnp.zeros((s_ref.shape[0],), jnp.float32)
    q = jnp.zeros((s_ref.shape[0],), jnp.float32)
    for k in range(K):
        h = jnp.dot(w1a, ed[k * c:(k + 1) * c, :],
                    preferred_element_type=jnp.float32) + u
        s = s + jnp.sum(h, axis=1)
        q = q + jnp.sum(h * h, axis=1)
    s_ref[...] += s[:, None]
    q_ref[...] += q[:, None]


def _affine(s, q, g, b, count):
    mu = s / count
    var = q / count - mu * mu
    sc = g * jax.lax.rsqrt(var + _EPS)
    sh = b - mu * sc
    return sc, sh


def _stats2_kernel(ed_ref, xtile_ref, w1_ref, w2_ref, s1_ref, q1_ref,
                   g1_ref, b1_ref, s_ref, q_ref, *, K, count):
    @pl.when((pl.program_id(0) == 0) & (pl.program_id(1) == 0))
    def _init():
        s_ref[...] = jnp.zeros_like(s_ref)
        q_ref[...] = jnp.zeros_like(q_ref)

    ed = ed_ref[0]
    xi = xtile_ref[0]
    c = xi.shape[0]
    w1a = w1_ref[:, :c]
    w1b = w1_ref[:, c:]
    sc1, sh1 = _affine(s1_ref[...], q1_ref[...], g1_ref[...], b1_ref[...],
                       count)
    u = jnp.dot(w1b, xi, preferred_element_type=jnp.float32)
    s = jnp.zeros((s_ref.shape[0],), jnp.float32)
    q = jnp.zeros((s_ref.shape[0],), jnp.float32)
    for k in range(K):
        h = jnp.dot(w1a, ed[k * c:(k + 1) * c, :],
                    preferred_element_type=jnp.float32) + u
        r = jnp.maximum(h * sc1 + sh1, 0.0)
        h2 = jnp.dot(w2_ref[...], r, preferred_element_type=jnp.float32)
        s = s + jnp.sum(h2, axis=1)
        q = q + jnp.sum(h2 * h2, axis=1)
    s_ref[...] += s[:, None]
    q_ref[...] += q[:, None]


def _final_kernel(ed_ref, xtile_ref, w1_ref, w2_ref, s1_ref, q1_ref,
                  g1_ref, b1_ref, s2_ref, q2_ref, g2_ref, b2_ref,
                  out_ref, *, K, count):
    ed = ed_ref[0]
    xi = xtile_ref[0]
    c = xi.shape[0]
    w1a = w1_ref[:, :c]
    w1b = w1_ref[:, c:]
    sc1, sh1 = _affine(s1_ref[...], q1_ref[...], g1_ref[...], b1_ref[...],
                       count)
    sc2, sh2 = _affine(s2_ref[...], q2_ref[...], g2_ref[...], b2_ref[...],
                       count)
    u = jnp.dot(w1b, xi, preferred_element_type=jnp.float32)
    acc = jnp.full((out_ref.shape[1], xi.shape[1]), _NEG, jnp.float32)
    for k in range(K):
        h = jnp.dot(w1a, ed[k * c:(k + 1) * c, :],
                    preferred_element_type=jnp.float32) + u
        r = jnp.maximum(h * sc1 + sh1, 0.0)
        h2 = jnp.dot(w2_ref[...], r, preferred_element_type=jnp.float32)
        acc = jnp.maximum(acc, h2 * sc2 + sh2)
    out_ref[0] = jnp.maximum(acc, 0.0)


def kernel(x, W1, g1, b1, W2, g2, b2):
    B, C, N = x.shape
    O = W1.shape[0]
    K = _K
    T1 = 128
    T2 = 512
    par = pltpu.CompilerParams(dimension_semantics=("arbitrary", "arbitrary"))
    f32 = jnp.float32
    g1c = g1.reshape(O, 1)
    b1c = b1.reshape(O, 1)
    g2c = g2.reshape(O, 1)
    b2c = b2.reshape(O, 1)
    count = float(B * N * K)

    ed = pl.pallas_call(
        functools.partial(_knn_edge_kernel, T=T1, N=N, K=K),
        grid=(B, N // T1),
        in_specs=[
            pl.BlockSpec((1, C, N), lambda b, t: (b, 0, 0)),
            pl.BlockSpec((1, C, T1), lambda b, t: (b, 0, t)),
        ],
        out_specs=pl.BlockSpec((1, C * K, T1), lambda b, t: (b, 0, t)),
        out_shape=jax.ShapeDtypeStruct((B, C * K, N), f32),
        compiler_params=par,
    )(x, x)

    vec_spec = pl.BlockSpec((O, 1), lambda b, t: (0, 0))
    w1_spec = pl.BlockSpec((O, 2 * C), lambda b, t: (0, 0))
    w2_spec = pl.BlockSpec((O, O), lambda b, t: (0, 0))
    ed_spec = pl.BlockSpec((1, C * K, T2), lambda b, t: (b, 0, t))
    xt_spec = pl.BlockSpec((1, C, T2), lambda b, t: (b, 0, t))

    s1, q1 = pl.pallas_call(
        functools.partial(_stats1_kernel, K=K),
        grid=(B, N // T2),
        in_specs=[ed_spec, xt_spec, w1_spec],
        out_specs=[vec_spec, vec_spec],
        out_shape=[jax.ShapeDtypeStruct((O, 1), f32)] * 2,
    )(ed, x, W1)

    s2, q2 = pl.pallas_call(
        functools.partial(_stats2_kernel, K=K, count=count),
        grid=(B, N // T2),
        in_specs=[ed_spec, xt_spec, w1_spec, w2_spec,
                  vec_spec, vec_spec, vec_spec, vec_spec],
        out_specs=[vec_spec, vec_spec],
        out_shape=[jax.ShapeDtypeStruct((O, 1), f32)] * 2,
    )(ed, x, W1, W2, s1, q1, g1c, b1c)

    out = pl.pallas_call(
        functools.partial(_final_kernel, K=K, count=count),
        grid=(B, N // T2),
        in_specs=[ed_spec, xt_spec, w1_spec, w2_spec,
                  vec_spec, vec_spec, vec_spec, vec_spec,
                  vec_spec, vec_spec, vec_spec, vec_spec],
        out_specs=pl.BlockSpec((1, O, T2), lambda b, t: (b, 0, t)),
        out_shape=jax.ShapeDtypeStruct((B, O, N), f32),
        compiler_params=par,
    )(ed, x, W1, W2, s1, q1, g1c, b1c, s2, q2, g2c, b2c)

    return out
